# baseline (speedup 1.0000x reference)
import functools

import jax
import jax.numpy as jnp
from jax import lax
from jax.experimental import pallas as pl
from jax.experimental.pallas import tpu as pltpu
from jax.experimental.pallas import tpu_sc as plsc

B = 8
H = 360
W = 357
SINSZ = H * W
NPIX = 65536
OUT = 256
KLEN = 713
NNZ = 2000000

NCORES = 2
NSUB = 16
NW = NCORES * NSUB
CB = 2000
NBLK = NNZ // CB
FULL_ROUNDS = NBLK // NW
LEFTOVER = NBLK - FULL_ROUNDS * NW


def _filter_body(sin_ref, cos_ref, m_ref, b_ref, out_ref):
    r = sin_ref[...] * cos_ref[...][None]
    x = r.reshape(B * H, W)
    y = lax.dot_general(x, m_ref[...], (((1,), (1,)), ((), ())),
                        preferred_element_type=jnp.float32)
    out_ref[...] = y + b_ref[0]


def _combine_body(p_ref, out_ref):
    x = p_ref[...]
    out_ref[...] = jnp.clip(x[0] + x[1], 0.0, 1.0)


def _spmm_sc(s_tab, rows, cols, data, zeros):
    mesh = plsc.VectorSubcoreMesh(core_axis_name="c", subcore_axis_name="s",
                                  num_cores=NCORES, num_subcores=NSUB)
    NSTEP = FULL_ROUNDS + 1

    @functools.partial(
        pl.kernel,
        out_type=jax.ShapeDtypeStruct((NCORES * NPIX, B), jnp.float32),
        mesh=mesh,
        compiler_params=pltpu.CompilerParams(needs_layout_passes=False,
                                             use_tc_tiling_on_sc=False),
        scratch_types=[
            pltpu.VMEM_SHARED((NPIX, B), jnp.float32),
            pltpu.VMEM((2, CB), jnp.int32),
            pltpu.VMEM((4, CB), jnp.int32),
            pltpu.VMEM((2, CB), jnp.float32),
            pltpu.VMEM((2, CB, B), jnp.float32),
            pltpu.VMEM((2, CB, B), jnp.float32),
            pltpu.SemaphoreType.DMA,
            pltpu.SemaphoreType.DMA,
            pltpu.SemaphoreType.DMA,
            pltpu.SemaphoreType.DMA,
            pltpu.SemaphoreType.DMA,
            pltpu.SemaphoreType.DMA,
        ],
    )
    def spmm_kernel(s_hbm, rows_hbm, cols_hbm, data_hbm, z_hbm, out_hbm,
                    acc, col_v, row_v, dat_v, gath_v, scl_v,
                    sem_i0, sem_i1, sem_g0, sem_g1, sem_s0, sem_s1):
        cid = lax.axis_index("c")
        sid = lax.axis_index("s")
        wid = cid * NSUB + sid
        rows_per = NPIX // NSUB
        sem_i = (sem_i0, sem_i1)
        sem_g = (sem_g0, sem_g1)
        sem_s = (sem_s0, sem_s1)

        io = lax.iota(jnp.int32, 16)
        hi = io // 8
        colg = io % 8

        def block_id(k):
            return jnp.where(k < FULL_ROUNDS, k * NW + wid,
                             jnp.where(wid < LEFTOVER,
                                       FULL_ROUNDS * NW + wid, wid))

        def idx_copies(k, p, q):
            base = block_id(k) * CB
            return (
                pltpu.make_async_copy(cols_hbm.at[pl.ds(base, CB)],
                                      col_v.at[p], sem_i[p]),
                pltpu.make_async_copy(rows_hbm.at[pl.ds(base, CB)],
                                      row_v.at[q], sem_i[p]),
                pltpu.make_async_copy(data_hbm.at[pl.ds(base, CB)],
                                      dat_v.at[p], sem_i[p]),
            )

        def gather_copy(p):
            return pltpu.make_async_copy(s_hbm.at[col_v.at[p]],
                                         gath_v.at[p], sem_g[p])

        def scatter_start(p, q):
            pltpu.async_copy(scl_v.at[p], acc.at[row_v.at[q]], sem_s[p],
                             add=True)

        def scatter_wait(p, q):
            pltpu.make_async_copy(scl_v.at[p],
                                  acc.at[row_v.at[q]], sem_s[p]).wait()

        def issue_idx(k, p, q):
            for c in idx_copies(k, p, q):
                c.start()

        def wait_idx(k, p, q):
            for c in idx_copies(k, p, q):
                c.wait()

        issue_idx(0, 0, 0)
        issue_idx(1, 1, 1)
        wait_idx(0, 0, 0)
        gather_copy(0).start()

        pltpu.sync_copy(z_hbm.at[pl.ds(sid * rows_per, rows_per)],
                        acc.at[pl.ds(sid * rows_per, rows_per)])
        plsc.subcore_barrier()

        def step(k, a, q):
            b = 1 - a
            @pl.when(k < NSTEP - 1)
            def _():
                wait_idx(k + 1, b, (q + 1) % 4)
                gather_copy(b).start()

            gather_copy(a).wait()

            @pl.when(k >= 2)
            def _():
                scatter_wait(a, (q + 2) % 4)

            valid = jnp.logical_or(k < FULL_ROUNDS, wid < LEFTOVER)
            f = jnp.where(valid, 1.0, 0.0).astype(jnp.float32)

            @pl.loop(0, CB // 2, unroll=8)
            def _(g):
                rowg = g * 2 + hi
                dv = plsc.load_gather(dat_v.at[a], [rowg])
                sv = plsc.load_gather(gath_v.at[a], [rowg, colg])
                plsc.store_scatter(scl_v.at[a], [rowg, colg], sv * dv * f)

            scatter_start(a, q)

            @pl.when(k < NSTEP - 2)
            def _():
                issue_idx(k + 2, a, (q + 2) % 4)

        @pl.loop(0, NSTEP // 4)
        def _(r):
            for j in range(4):
                step(r * 4 + j, j % 2, j)

        scatter_wait(0, 2)
        scatter_wait(1, 3)

        plsc.subcore_barrier()
        pltpu.sync_copy(
            acc.at[pl.ds(sid * rows_per, rows_per)],
            out_hbm.at[pl.ds(cid * NPIX + sid * rows_per, rows_per)])

    return spmm_kernel(s_tab, rows, cols, data, zeros)


def kernel(sin_fan, cos, filt_w, filt_b, A_rows, A_cols, A_data):
    vp = jnp.concatenate([filt_w, jnp.zeros((1,), jnp.float32)])
    flat = jnp.broadcast_to(vp, (W, KLEN + 1)).reshape(-1)
    m = flat[: W * KLEN].reshape(W, KLEN)[:, (KLEN - 1) // 2:]

    y = pl.pallas_call(
        _filter_body,
        out_shape=jax.ShapeDtypeStruct((B * H, W), jnp.float32),
    )(sin_fan, cos, m, filt_b)

    s_tab = jnp.transpose(y.reshape(B, H, W), (1, 2, 0)).reshape(SINSZ, B)

    zeros = jnp.zeros((NPIX, B), jnp.float32)
    parts = _spmm_sc(s_tab, A_rows, A_cols, A_data, zeros)

    fbp = pl.pallas_call(
        _combine_body,
        out_shape=jax.ShapeDtypeStruct((NPIX * B,), jnp.float32),
    )(parts.reshape(NCORES, NPIX * B))

    return fbp.reshape(NPIX, B).T.reshape(B, OUT, OUT, 1)


# speedup vs baseline: 1.1008x; 1.1008x over previous
import functools

import jax
import jax.numpy as jnp
from jax import lax
from jax.experimental import pallas as pl
from jax.experimental.pallas import tpu as pltpu
from jax.experimental.pallas import tpu_sc as plsc

B = 8
H = 360
W = 357
SINSZ = H * W
NPIX = 65536
OUT = 256
KLEN = 713
NNZ = 2000000

NCORES = 2
NSUB = 16
NW = NCORES * NSUB
CB = 2000
NBLK = NNZ // CB
FULL_ROUNDS = NBLK // NW
LEFTOVER = NBLK - FULL_ROUNDS * NW

SINSZ_P = 131072
TC_PX = 512
TC_CHUNKS = SINSZ_P // TC_PX
TC_FULL = SINSZ // TC_PX


def _filter_body(sin_ref, cos_ref, m_ref, b_ref, out_ref):
    r = sin_ref[...] * cos_ref[...][None]
    x = r.reshape(B * H, W)
    y = lax.dot_general(x, m_ref[...], (((1,), (1,)), ((), ())),
                        preferred_element_type=jnp.float32)
    out_ref[...] = y + b_ref[0]


def _combine_body(p_ref, out_ref):
    x = p_ref[...]
    out_ref[...] = jnp.clip(x[0] + x[1], 0.0, 1.0)


def _spmm_sc(y2, rows, cols, data, zeros):
    mesh = plsc.VectorSubcoreMesh(core_axis_name="c", subcore_axis_name="s",
                                  num_cores=NCORES, num_subcores=NSUB)
    NSTEP = FULL_ROUNDS + 1

    @functools.partial(
        pl.kernel,
        out_type=(jax.ShapeDtypeStruct((NCORES * NPIX, B), jnp.float32),
                  jax.ShapeDtypeStruct((SINSZ_P, B), jnp.float32)),
        mesh=mesh,
        compiler_params=pltpu.CompilerParams(needs_layout_passes=False,
                                             use_tc_tiling_on_sc=False),
        scratch_types=[
            pltpu.VMEM_SHARED((NPIX, B), jnp.float32),
            pltpu.VMEM((2, CB), jnp.int32),
            pltpu.VMEM((4, CB), jnp.int32),
            pltpu.VMEM((2, CB), jnp.float32),
            pltpu.VMEM((2, CB, B), jnp.float32),
            pltpu.VMEM((2, CB, B), jnp.float32),
            pltpu.VMEM((B, TC_PX), jnp.float32),
            pltpu.VMEM((TC_PX, B), jnp.float32),
            pltpu.SemaphoreType.DMA,
            pltpu.SemaphoreType.DMA,
            pltpu.SemaphoreType.DMA,
            pltpu.SemaphoreType.DMA,
            pltpu.SemaphoreType.DMA,
            pltpu.SemaphoreType.DMA,
        ],
    )
    def spmm_kernel(y2_hbm, rows_hbm, cols_hbm, data_hbm, z_hbm,
                    out_hbm, stab_hbm,
                    acc, col_v, row_v, dat_v, gath_v, scl_v,
                    ybuf, tbuf,
                    sem_i0, sem_i1, sem_g0, sem_g1, sem_s0, sem_s1):
        cid = lax.axis_index("c")
        sid = lax.axis_index("s")
        wid = cid * NSUB + sid
        rows_per = NPIX // NSUB
        sem_i = (sem_i0, sem_i1)
        sem_g = (sem_g0, sem_g1)
        sem_s = (sem_s0, sem_s1)

        io = lax.iota(jnp.int32, 16)
        hi = io // 8
        colg = io % 8

        def block_id(k):
            return jnp.where(k < FULL_ROUNDS, k * NW + wid,
                             jnp.where(wid < LEFTOVER,
                                       FULL_ROUNDS * NW + wid, wid))

        def idx_copies(k, p, q):
            base = block_id(k) * CB
            return (
                pltpu.make_async_copy(cols_hbm.at[pl.ds(base, CB)],
                                      col_v.at[p], sem_i[p]),
                pltpu.make_async_copy(rows_hbm.at[pl.ds(base, CB)],
                                      row_v.at[q], sem_i[p]),
                pltpu.make_async_copy(data_hbm.at[pl.ds(base, CB)],
                                      dat_v.at[p], sem_i[p]),
            )

        def gather_copy(p):
            return pltpu.make_async_copy(stab_hbm.at[col_v.at[p]],
                                         gath_v.at[p], sem_g[p])

        def scatter_start(p, q):
            pltpu.async_copy(scl_v.at[p], acc.at[row_v.at[q]], sem_s[p],
                             add=True)

        def scatter_wait(p, q):
            pltpu.make_async_copy(scl_v.at[p],
                                  acc.at[row_v.at[q]], sem_s[p]).wait()

        def issue_idx(k, p, q):
            for c in idx_copies(k, p, q):
                c.start()

        def wait_idx(k, p, q):
            for c in idx_copies(k, p, q):
                c.wait()

        issue_idx(0, 0, 0)
        issue_idx(1, 1, 1)

        @pl.loop(0, TC_CHUNKS // NSUB)
        def _(r):
            chunk = r * NSUB + sid
            p0 = chunk * TC_PX

            @pl.when(chunk < TC_FULL)
            def _():
                pltpu.sync_copy(y2_hbm.at[:, pl.ds(p0, TC_PX)], ybuf)

            @pl.when(chunk == TC_FULL)
            def _():
                pltpu.sync_copy(y2_hbm.at[:, pl.ds(SINSZ - 8, 8)],
                                ybuf.at[:, pl.ds(0, 8)])

            @pl.when(chunk <= TC_FULL)
            def _():
                for b in range(B):
                    bcol = jnp.full((16,), b, jnp.int32)

                    @pl.loop(0, TC_PX // 16)
                    def _(g):
                        v = ybuf[b, pl.ds(g * 16, 16)]
                        plsc.store_scatter(tbuf, [g * 16 + io, bcol], v)
                pltpu.sync_copy(tbuf, stab_hbm.at[pl.ds(p0, TC_PX)])

        pltpu.sync_copy(z_hbm,
                        acc.at[pl.ds(sid * rows_per, rows_per)])
        plsc.subcore_barrier()
        wait_idx(0, 0, 0)
        gather_copy(0).start()

        def step(k, a, q):
            b = 1 - a
            @pl.when(k < NSTEP - 1)
            def _():
                wait_idx(k + 1, b, (q + 1) % 4)
                gather_copy(b).start()

            gather_copy(a).wait()

            @pl.when(k >= 2)
            def _():
                scatter_wait(a, (q + 2) % 4)

            valid = jnp.logical_or(k < FULL_ROUNDS, wid < LEFTOVER)
            f = jnp.where(valid, 1.0, 0.0).astype(jnp.float32)

            @pl.loop(0, CB // 2, unroll=8)
            def _(g):
                rowg = g * 2 + hi
                dv = plsc.load_gather(dat_v.at[a], [rowg])
                sv = plsc.load_gather(gath_v.at[a], [rowg, colg])
                plsc.store_scatter(scl_v.at[a], [rowg, colg], sv * dv * f)

            scatter_start(a, q)

            @pl.when(k < NSTEP - 2)
            def _():
                issue_idx(k + 2, a, (q + 2) % 4)

        @pl.loop(0, NSTEP // 4)
        def _(r):
            for j in range(4):
                step(r * 4 + j, j % 2, j)

        scatter_wait(0, 2)
        scatter_wait(1, 3)

        plsc.subcore_barrier()
        pltpu.sync_copy(
            acc.at[pl.ds(sid * rows_per, rows_per)],
            out_hbm.at[pl.ds(cid * NPIX + sid * rows_per, rows_per)])

    parts, _ = spmm_kernel(y2, rows, cols, data, zeros)
    return parts


def kernel(sin_fan, cos, filt_w, filt_b, A_rows, A_cols, A_data):
    vp = jnp.concatenate([filt_w, jnp.zeros((1,), jnp.float32)])
    flat = jnp.broadcast_to(vp, (W, KLEN + 1)).reshape(-1)
    m = flat[: W * KLEN].reshape(W, KLEN)[:, (KLEN - 1) // 2:]

    y = pl.pallas_call(
        _filter_body,
        out_shape=jax.ShapeDtypeStruct((B * H, W), jnp.float32),
    )(sin_fan, cos, m, filt_b)

    y2 = y.reshape(B, SINSZ)

    zeros = jnp.zeros((NPIX // NSUB, B), jnp.float32)
    parts = _spmm_sc(y2, A_rows, A_cols, A_data, zeros)

    fbp = pl.pallas_call(
        _combine_body,
        out_shape=jax.ShapeDtypeStruct((NPIX * B,), jnp.float32),
    )(parts.reshape(NCORES, NPIX * B))

    return fbp.reshape(NPIX, B).T.reshape(B, OUT, OUT, 1)


# speedup vs baseline: 1.2193x; 1.1077x over previous
import functools

import jax
import jax.numpy as jnp
from jax import lax
from jax.experimental import pallas as pl
from jax.experimental.pallas import tpu as pltpu
from jax.experimental.pallas import tpu_sc as plsc

B = 8
H = 360
W = 357
SINSZ = H * W
NPIX = 65536
OUT = 256
KLEN = 713
NNZ = 2000000

NCORES = 2
NSUB = 16
NW = NCORES * NSUB
CB = 2000
NBLK = NNZ // CB
FULL_ROUNDS = NBLK // NW
LEFTOVER = NBLK - FULL_ROUNDS * NW

SINSZ_P = 131072
TC_PX = 512
TC_CHUNKS = SINSZ_P // TC_PX
TC_FULL = SINSZ // TC_PX


def _filter_body(sin_ref, cos_ref, m_ref, b_ref, out_ref):
    r = sin_ref[...] * cos_ref[...][None]
    x = r.reshape(B * H, W)
    y = lax.dot_general(x, m_ref[...], (((1,), (1,)), ((), ())),
                        preferred_element_type=jnp.float32)
    out_ref[...] = y + b_ref[0]


def _combine_sc(parts_flat):
    mesh = plsc.VectorSubcoreMesh(core_axis_name="c", subcore_axis_name="s",
                                  num_cores=NCORES, num_subcores=NSUB)
    PC = NPIX // NW

    @functools.partial(
        pl.kernel,
        out_type=jax.ShapeDtypeStruct((B, NPIX), jnp.float32),
        mesh=mesh,
        compiler_params=pltpu.CompilerParams(needs_layout_passes=False,
                                             use_tc_tiling_on_sc=False),
        scratch_types=[
            pltpu.VMEM((PC * B,), jnp.float32),
            pltpu.VMEM((PC * B,), jnp.float32),
            pltpu.VMEM((B, PC), jnp.float32),
        ],
    )
    def combine_kernel(p_hbm, out_hbm, a0, a1, ob):
        cid = lax.axis_index("c")
        sid = lax.axis_index("s")
        wid = cid * NSUB + sid
        p0 = wid * PC

        pltpu.sync_copy(p_hbm.at[pl.ds(p0 * B, PC * B)], a0)
        pltpu.sync_copy(p_hbm.at[pl.ds((NPIX + p0) * B, PC * B)], a1)

        io = lax.iota(jnp.int32, 16)
        for b in range(B):
            idx0 = io * B + b

            @pl.loop(0, PC // 16, unroll=8)
            def _(g):
                idx = g * (16 * B) + idx0
                v = plsc.load_gather(a0, [idx]) + plsc.load_gather(a1, [idx])
                ob[b, pl.ds(g * 16, 16)] = jnp.clip(v, 0.0, 1.0)

        pltpu.sync_copy(ob, out_hbm.at[:, pl.ds(p0, PC)])

    return combine_kernel(parts_flat)


def _spmm_sc(y2, rows, cols, data, zeros):
    mesh = plsc.VectorSubcoreMesh(core_axis_name="c", subcore_axis_name="s",
                                  num_cores=NCORES, num_subcores=NSUB)
    NSTEP = FULL_ROUNDS + 1

    @functools.partial(
        pl.kernel,
        out_type=(jax.ShapeDtypeStruct((NCORES * NPIX, B), jnp.float32),
                  jax.ShapeDtypeStruct((SINSZ_P, B), jnp.float32)),
        mesh=mesh,
        compiler_params=pltpu.CompilerParams(needs_layout_passes=False,
                                             use_tc_tiling_on_sc=False),
        scratch_types=[
            pltpu.VMEM_SHARED((NPIX, B), jnp.float32),
            pltpu.VMEM((2, CB), jnp.int32),
            pltpu.VMEM((4, CB), jnp.int32),
            pltpu.VMEM((2, CB), jnp.float32),
            pltpu.VMEM((2, CB, B), jnp.float32),
            pltpu.VMEM((2, CB, B), jnp.float32),
            pltpu.VMEM((B, TC_PX), jnp.float32),
            pltpu.VMEM((TC_PX, B), jnp.float32),
            pltpu.SemaphoreType.DMA,
            pltpu.SemaphoreType.DMA,
            pltpu.SemaphoreType.DMA,
            pltpu.SemaphoreType.DMA,
            pltpu.SemaphoreType.DMA,
            pltpu.SemaphoreType.DMA,
        ],
    )
    def spmm_kernel(y2_hbm, rows_hbm, cols_hbm, data_hbm, z_hbm,
                    out_hbm, stab_hbm,
                    acc, col_v, row_v, dat_v, gath_v, scl_v,
                    ybuf, tbuf,
                    sem_i0, sem_i1, sem_g0, sem_g1, sem_s0, sem_s1):
        cid = lax.axis_index("c")
        sid = lax.axis_index("s")
        wid = cid * NSUB + sid
        rows_per = NPIX // NSUB
        sem_i = (sem_i0, sem_i1)
        sem_g = (sem_g0, sem_g1)
        sem_s = (sem_s0, sem_s1)

        io = lax.iota(jnp.int32, 16)
        hi = io // 8
        colg = io % 8

        def block_id(k):
            return jnp.where(k < FULL_ROUNDS, k * NW + wid,
                             jnp.where(wid < LEFTOVER,
                                       FULL_ROUNDS * NW + wid, wid))

        def idx_copies(k, p, q):
            base = block_id(k) * CB
            return (
                pltpu.make_async_copy(cols_hbm.at[pl.ds(base, CB)],
                                      col_v.at[p], sem_i[p]),
                pltpu.make_async_copy(rows_hbm.at[pl.ds(base, CB)],
                                      row_v.at[q], sem_i[p]),
                pltpu.make_async_copy(data_hbm.at[pl.ds(base, CB)],
                                      dat_v.at[p], sem_i[p]),
            )

        def gather_copy(p):
            return pltpu.make_async_copy(stab_hbm.at[col_v.at[p]],
                                         gath_v.at[p], sem_g[p])

        def scatter_start(p, q):
            pltpu.async_copy(scl_v.at[p], acc.at[row_v.at[q]], sem_s[p],
                             add=True)

        def scatter_wait(p, q):
            pltpu.make_async_copy(scl_v.at[p],
                                  acc.at[row_v.at[q]], sem_s[p]).wait()

        def issue_idx(k, p, q):
            for c in idx_copies(k, p, q):
                c.start()

        def wait_idx(k, p, q):
            for c in idx_copies(k, p, q):
                c.wait()

        issue_idx(0, 0, 0)
        issue_idx(1, 1, 1)

        @pl.loop(0, TC_CHUNKS // NSUB)
        def _(r):
            chunk = r * NSUB + sid
            p0 = chunk * TC_PX

            @pl.when(chunk < TC_FULL)
            def _():
                pltpu.sync_copy(y2_hbm.at[:, pl.ds(p0, TC_PX)], ybuf)

            @pl.when(chunk == TC_FULL)
            def _():
                pltpu.sync_copy(y2_hbm.at[:, pl.ds(SINSZ - 8, 8)],
                                ybuf.at[:, pl.ds(0, 8)])

            @pl.when(chunk <= TC_FULL)
            def _():
                for b in range(B):
                    bcol = jnp.full((16,), b, jnp.int32)

                    @pl.loop(0, TC_PX // 16)
                    def _(g):
                        v = ybuf[b, pl.ds(g * 16, 16)]
                        plsc.store_scatter(tbuf, [g * 16 + io, bcol], v)
                pltpu.sync_copy(tbuf, stab_hbm.at[pl.ds(p0, TC_PX)])

        pltpu.sync_copy(z_hbm,
                        acc.at[pl.ds(sid * rows_per, rows_per)])
        plsc.subcore_barrier()
        wait_idx(0, 0, 0)
        gather_copy(0).start()

        def step(k, a, q):
            b = 1 - a
            @pl.when(k < NSTEP - 1)
            def _():
                wait_idx(k + 1, b, (q + 1) % 4)
                gather_copy(b).start()

            gather_copy(a).wait()

            @pl.when(k >= 2)
            def _():
                scatter_wait(a, (q + 2) % 4)

            valid = jnp.logical_or(k < FULL_ROUNDS, wid < LEFTOVER)
            f = jnp.where(valid, 1.0, 0.0).astype(jnp.float32)

            @pl.loop(0, CB // 2, unroll=8)
            def _(g):
                rowg = g * 2 + hi
                dv = plsc.load_gather(dat_v.at[a], [rowg])
                sv = plsc.load_gather(gath_v.at[a], [rowg, colg])
                plsc.store_scatter(scl_v.at[a], [rowg, colg], sv * dv * f)

            scatter_start(a, q)

            @pl.when(k < NSTEP - 2)
            def _():
                issue_idx(k + 2, a, (q + 2) % 4)

        @pl.loop(0, NSTEP // 4)
        def _(r):
            for j in range(4):
                step(r * 4 + j, j % 2, j)

        scatter_wait(0, 2)
        scatter_wait(1, 3)

        plsc.subcore_barrier()
        pltpu.sync_copy(
            acc.at[pl.ds(sid * rows_per, rows_per)],
            out_hbm.at[pl.ds(cid * NPIX + sid * rows_per, rows_per)])

    parts, _ = spmm_kernel(y2, rows, cols, data, zeros)
    return parts


def kernel(sin_fan, cos, filt_w, filt_b, A_rows, A_cols, A_data):
    vp = jnp.concatenate([filt_w, jnp.zeros((1,), jnp.float32)])
    flat = jnp.broadcast_to(vp, (W, KLEN + 1)).reshape(-1)
    m = flat[: W * KLEN].reshape(W, KLEN)[:, (KLEN - 1) // 2:]

    y = pl.pallas_call(
        _filter_body,
        out_shape=jax.ShapeDtypeStruct((B * H, W), jnp.float32),
    )(sin_fan, cos, m, filt_b)

    y2 = y.reshape(B, SINSZ)

    zeros = jnp.zeros((NPIX // NSUB, B), jnp.float32)
    parts = _spmm_sc(y2, A_rows, A_cols, A_data, zeros)

    fbp = _combine_sc(parts.reshape(NCORES * NPIX * B))
    return fbp.reshape(B, OUT, OUT, 1)
